# Initial kernel scaffold; baseline (speedup 1.0000x reference)
import jax
import jax.numpy as jnp
from jax.experimental import pallas as pl


def kernel(x, edge_index, edge_attr, batch, W0, b0, eps, W1s, b1s, g1s, be1s, W2s, b2s, gs, bes, Wout, bout):
    raise NotImplementedError("write your pallas kernel here")


# speedup vs baseline: 4.2724x; 4.2724x over previous
import functools

import jax
import jax.numpy as jnp
from jax import lax
from jax.experimental import pallas as pl
from jax.experimental.pallas import tpu as pltpu
from jax.experimental.pallas import tpu_sc as plsc

N = 10000
NP = 10112
E = 320000
FEAT = 128
G = 128
NC = 2
NS = 16
NW = NC * NS
CHUNK = 128
NCH = (E + NW * CHUNK - 1) // (NW * CHUNK)
EP = NW * NCH * CHUNK
RPS = NP // NS


def _sc_agg_body(h_hbm, src_hbm, dst_hbm, out_hbm,
                 src_v, dst_v, rows_v, zv, acc_sh, sem):
    c = lax.axis_index("c")
    s = lax.axis_index("s")
    wid = s * NC + c

    pltpu.sync_copy(src_hbm.at[wid], src_v)
    pltpu.sync_copy(dst_hbm.at[wid], dst_v)

    zero16 = jnp.zeros((16,), jnp.float32)
    for r in range(16):
        for q in range(8):
            zv[r, pl.ds(q * 16, 16)] = zero16
    row0 = s * RPS

    @pl.loop(0, 39)
    def _zero(i):
        pltpu.sync_copy(zv, acc_sh.at[pl.ds(row0 + i * 16, 16)])

    pltpu.sync_copy(zv.at[pl.ds(0, 8)], acc_sh.at[pl.ds(row0 + 624, 8)])
    plsc.subcore_barrier()

    @pl.loop(0, NCH)
    def _edge(j):
        pltpu.async_copy(h_hbm.at[src_v.at[j]], rows_v, sem).wait()
        pltpu.sync_copy(rows_v, acc_sh.at[dst_v.at[j]], add=True)

    plsc.subcore_barrier()
    pltpu.sync_copy(acc_sh.at[pl.ds(row0, RPS)],
                    out_hbm.at[c, pl.ds(row0, RPS)])


_sc_agg = functools.partial(
    pl.kernel,
    out_type=jax.ShapeDtypeStruct((NC, NP, FEAT), jnp.float32),
    mesh=plsc.VectorSubcoreMesh(core_axis_name="c", subcore_axis_name="s"),
    scratch_types=[
        pltpu.VMEM((NCH, CHUNK), jnp.int32),
        pltpu.VMEM((NCH, CHUNK), jnp.int32),
        pltpu.VMEM((CHUNK, FEAT), jnp.float32),
        pltpu.VMEM((16, FEAT), jnp.float32),
        pltpu.VMEM_SHARED((NP, FEAT), jnp.float32),
        pltpu.SemaphoreType.DMA,
    ],
)(_sc_agg_body)


def _tc_input_body(x_ref, w_ref, b_ref, o_ref):
    h = jnp.dot(x_ref[...], w_ref[...], preferred_element_type=jnp.float32)
    o_ref[0:N, :] = jnp.maximum(h + b_ref[...], 0.0)


def _tc_input(x, w0, b0):
    return pl.pallas_call(
        _tc_input_body,
        out_shape=jax.ShapeDtypeStruct((NP, FEAT), jnp.float32),
    )(x, w0, b0)


def _bn_relu(z, g, b):
    m = jnp.sum(z, axis=0, keepdims=True) * (1.0 / N)
    d = z - m
    v = jnp.sum(d * d, axis=0, keepdims=True) * (1.0 / N)
    return jnp.maximum(g * d * jax.lax.rsqrt(v + 1e-5) + b, 0.0)


def _tc_dense_body(h_ref, p_ref, eps_ref, w1_ref, b1_ref, g1_ref, be1_ref,
                   w2_ref, b2_ref, g2_ref, be2_ref, o_ref):
    h = h_ref[0:N, :]
    agg = p_ref[0, 0:N, :] + p_ref[1, 0:N, :]
    z = (1.0 + eps_ref[0, 0]) * h + agg
    z = jnp.dot(z, w1_ref[...], preferred_element_type=jnp.float32) + b1_ref[...]
    z = _bn_relu(z, g1_ref[...], be1_ref[...])
    z = jnp.dot(z, w2_ref[...], preferred_element_type=jnp.float32) + b2_ref[...]
    z = _bn_relu(z, g2_ref[...], be2_ref[...])
    o_ref[0:N, :] = z + h


def _tc_dense(h, parts, eps_l, w1, b1, g1, be1, w2, b2, g2, be2):
    return pl.pallas_call(
        _tc_dense_body,
        out_shape=jax.ShapeDtypeStruct((NP, FEAT), jnp.float32),
    )(h, parts, eps_l, w1, b1, g1, be1, w2, b2, g2, be2)


def _tc_pool_body(h_ref, batch_ref, wout_ref, bout_ref, o_ref):
    h = h_ref[0:N, :]
    ids = batch_ref[...]
    onehot = (ids == lax.broadcasted_iota(jnp.int32, (N, G), 1))
    onehot = onehot.astype(jnp.float32)
    cdims = (((0,), (0,)), ((), ()))
    sums = lax.dot_general(onehot, h, cdims,
                           preferred_element_type=jnp.float32)
    cnt = lax.dot_general(onehot, jnp.ones((N, 1), jnp.float32), cdims,
                          preferred_element_type=jnp.float32)
    pooled = sums / jnp.maximum(cnt, 1.0)
    o_ref[...] = jnp.dot(pooled, wout_ref[...],
                         preferred_element_type=jnp.float32) + bout_ref[...]


def _tc_pool(h, batch2d, wout, bout):
    return pl.pallas_call(
        _tc_pool_body,
        out_shape=jax.ShapeDtypeStruct((G, FEAT), jnp.float32),
    )(h, batch2d, wout, bout)


def kernel(x, edge_index, edge_attr, batch, W0, b0, eps, W1s, b1s, g1s, be1s,
           W2s, b2s, gs, bes, Wout, bout):
    pad = EP - E
    src = jnp.concatenate([edge_index[0], jnp.full((pad,), N, jnp.int32)])
    dst = jnp.concatenate([edge_index[1], jnp.full((pad,), N, jnp.int32)])
    src = src.reshape(NW, NCH, CHUNK)
    dst = dst.reshape(NW, NCH, CHUNK)

    h = _tc_input(x, W0, b0.reshape(1, FEAT))
    for l in range(2):
        parts = _sc_agg(h, src, dst)
        h = _tc_dense(h, parts, eps[l].reshape(1, 1),
                      W1s[l], b1s[l].reshape(1, FEAT),
                      g1s[l].reshape(1, FEAT), be1s[l].reshape(1, FEAT),
                      W2s[l], b2s[l].reshape(1, FEAT),
                      gs[l].reshape(1, FEAT), bes[l].reshape(1, FEAT))
    return _tc_pool(h, batch.reshape(N, 1).astype(jnp.int32),
                    Wout, bout.reshape(1, FEAT))
